# Initial kernel scaffold; baseline (speedup 1.0000x reference)
import jax
import jax.numpy as jnp
from jax.experimental import pallas as pl


def kernel(feat, edge_index, W):
    raise NotImplementedError("write your pallas kernel here")


# speedup vs baseline: 6.1696x; 6.1696x over previous
import functools

import jax
import jax.numpy as jnp
from jax import lax
from jax.experimental import pallas as pl
from jax.experimental.pallas import tpu as pltpu
from jax.experimental.pallas import tpu_sc as plsc

N = 10000
E = 320000
D = 128

NC = 2
NS = 16
NW = NC * NS
LANES = 16

E_PER_W = E // NW
CHUNK = 80
STEPS = E_PER_W // CHUNK
NP = 10240
ROWS_PER_TILE = NP // NS
ZROWS = 128


def _sc_aggregate(feat_hbm, src_hbm, dst_hbm, partial_hbm, deg_hbm,
                  idxs_v, idxd_v, rows_v, hist_v, zbuf_v, acc_sh, sem):
    c = lax.axis_index("c")
    s = lax.axis_index("s")
    wid = c * NS + s

    zeros16 = jnp.zeros((LANES,), jnp.float32)

    def zero_zbuf(k, _):
        i = k // (D // LANES)
        j = k % (D // LANES)
        zbuf_v[i, pl.ds(j * LANES, LANES)] = zeros16
        return 0
    lax.fori_loop(0, ZROWS * (D // LANES), zero_zbuf, 0)

    def zero_hist(k, _):
        hist_v[pl.ds(k * LANES, LANES)] = zeros16
        return 0
    lax.fori_loop(0, N // LANES, zero_hist, 0)

    for t in range(ROWS_PER_TILE // ZROWS):
        pltpu.sync_copy(zbuf_v, acc_sh.at[pl.ds(s * ROWS_PER_TILE + t * ZROWS, ZROWS)])

    plsc.subcore_barrier()

    ones16 = jnp.ones((LANES,), jnp.float32)

    def step(i, _):
        base = wid * E_PER_W + i * CHUNK
        pltpu.sync_copy(src_hbm.at[pl.ds(base, CHUNK)], idxs_v)
        pltpu.sync_copy(dst_hbm.at[pl.ds(base, CHUNK)], idxd_v)
        pltpu.async_copy(feat_hbm.at[idxs_v], rows_v, sem).wait()
        pltpu.sync_copy(rows_v, acc_sh.at[idxd_v], add=True)
        for j in range(CHUNK // LANES):
            idx = idxd_v[pl.ds(j * LANES, LANES)]
            plsc.addupdate_scatter(hist_v, [idx], ones16)
        return 0

    lax.fori_loop(0, STEPS, step, 0)

    plsc.subcore_barrier()

    for t in range(ROWS_PER_TILE // ZROWS):
        r0 = s * ROWS_PER_TILE + t * ZROWS
        pltpu.sync_copy(acc_sh.at[pl.ds(r0, ZROWS)], partial_hbm.at[c, pl.ds(r0, ZROWS)])
    pltpu.sync_copy(hist_v, deg_hbm.at[pl.ds(wid * N, N)])


_sc_call = functools.partial(
    pl.kernel,
    out_type=[
        jax.ShapeDtypeStruct((NC, NP, D), jnp.float32),
        jax.ShapeDtypeStruct((NW * N,), jnp.float32),
    ],
    mesh=plsc.VectorSubcoreMesh(core_axis_name="c", subcore_axis_name="s"),
    compiler_params=pltpu.CompilerParams(needs_layout_passes=False),
    scratch_types=[
        pltpu.VMEM((CHUNK,), jnp.int32),
        pltpu.VMEM((CHUNK,), jnp.int32),
        pltpu.VMEM((CHUNK, D), jnp.float32),
        pltpu.VMEM((N,), jnp.float32),
        pltpu.VMEM((ZROWS, D), jnp.float32),
        pltpu.VMEM_SHARED((NP, D), jnp.float32),
        pltpu.SemaphoreType.DMA,
    ],
)(_sc_aggregate)


ROWS_BLK = 400


def _tc_combine(partial_ref, deg_ref, w_ref, out_ref):
    p = partial_ref[...]
    summed = p[0] + p[1]
    deg = jnp.sum(deg_ref[...], axis=1)
    deg = jnp.maximum(deg, 1.0)
    mean = summed / deg[:, None]
    out_ref[...] = lax.dot_general(
        mean, w_ref[...], (((1,), (1,)), ((), ())),
        preferred_element_type=jnp.float32)


def _combine(partial, deg, W):
    return pl.pallas_call(
        _tc_combine,
        grid=(N // ROWS_BLK,),
        in_specs=[
            pl.BlockSpec((NC, ROWS_BLK, D), lambda i: (0, i, 0)),
            pl.BlockSpec((ROWS_BLK, NW), lambda i: (i, 0)),
            pl.BlockSpec((D, D), lambda i: (0, 0)),
        ],
        out_specs=pl.BlockSpec((ROWS_BLK, D), lambda i: (i, 0)),
        out_shape=jax.ShapeDtypeStruct((N, D), jnp.float32),
    )(partial, deg, W)


def kernel(feat, edge_index, W):
    src = edge_index[0]
    dst = edge_index[1]
    partial, deg = _sc_call(feat, src, dst)
    return _combine(partial, deg.reshape(NW, N).T, W)
